# baseline (speedup 1.0000x reference)
import jax
import jax.numpy as jnp
from jax import lax
from jax.experimental import pallas as pl
from jax.experimental.pallas import tpu as pltpu
from jax.experimental.pallas import tpu_sc as plsc

_LATENT = 64
_MU2 = 7.0
_B = 16384

_info = plsc.get_sparse_core_info()
_NC = _info.num_cores
_NS = _info.num_subcores
_NW = _NC * _NS
_BPW = _B // _NW
_L = _info.num_lanes
_IPAD = _BPW + _L
_D = 4


def _body(uidx_hbm, iidx_hbm, uembt_hbm, iembt_hbm, ubt_hbm, ibt_hbm,
          out_hbm,
          uidx_v, iidx_v,
          up0, up1, up2, up3, ip0, ip1, ip2, ip3,
          ub0, ub1, ub2, ub3, ib0, ib1, ib2, ib3,
          out_v,
          su0, su1, su2, su3, si0, si1, si2, si3,
          sb0, sb1, sb2, sb3):
    wid = lax.axis_index("s") * _NC + lax.axis_index("c")
    base = wid * _BPW

    upan = (up0, up1, up2, up3)
    ipan = (ip0, ip1, ip2, ip3)
    ubp = (ub0, ub1, ub2, ub3)
    ibp = (ib0, ib1, ib2, ib3)
    su = (su0, su1, su2, su3)
    si = (si0, si1, si2, si3)
    sb = (sb0, sb1, sb2, sb3)

    pltpu.sync_copy(uidx_hbm.at[pl.ds(base, _BPW)],
                    uidx_v.at[pl.ds(0, _BPW)])
    pltpu.sync_copy(iidx_hbm.at[pl.ds(base, _BPW)],
                    iidx_v.at[pl.ds(0, _BPW)])

    iota16 = lax.iota(jnp.int32, _L)
    zeros16 = jnp.zeros((_L,), jnp.int32)

    def zinit(g, c):
        out_v[pl.ds(g * _L, _L)] = jnp.zeros((_L,), jnp.float32)
        return c

    lax.fori_loop(0, _BPW // _L, zinit, 0)

    def fetch(r, s):
        ui = uidx_v[pl.ds(r, _L)][0]
        ii = iidx_v[pl.ds(r, _L)][0]
        uoff = pl.multiple_of((ui >> 7) * 128, 128)
        ioff = pl.multiple_of((ii >> 7) * 128, 128)
        pltpu.async_copy(uembt_hbm.at[:, pl.ds(uoff, 128)], upan[s], su[s])
        pltpu.async_copy(iembt_hbm.at[:, pl.ds(ioff, 128)], ipan[s], si[s])
        pltpu.async_copy(ubt_hbm.at[:, pl.ds(uoff, 128)], ubp[s], sb[s])
        pltpu.async_copy(ibt_hbm.at[:, pl.ds(ioff, 128)], ibp[s], sb[s])

    def wait(s):
        pltpu.make_async_copy(uembt_hbm.at[:, pl.ds(0, 128)], upan[s],
                              su[s]).wait()
        pltpu.make_async_copy(iembt_hbm.at[:, pl.ds(0, 128)], ipan[s],
                              si[s]).wait()
        pltpu.make_async_copy(ubt_hbm.at[:, pl.ds(0, 128)], ubp[s],
                              sb[s]).wait()
        pltpu.make_async_copy(ibt_hbm.at[:, pl.ds(0, 128)], ibp[s],
                              sb[s]).wait()

    def extract(r, s):
        ui = uidx_v[pl.ds(r, _L)][0]
        ii = iidx_v[pl.ds(r, _L)][0]
        ulane = zeros16 + (ui & 127)
        ilane = zeros16 + (ii & 127)
        part = jnp.zeros((_L,), jnp.float32)
        for c in range(_LATENT // _L):
            dvec = c * _L + iota16
            du = plsc.load_gather(upan[s], [dvec, ulane])
            di = plsc.load_gather(ipan[s], [dvec, ilane])
            part = part + du * di
        bu = plsc.load_gather(ubp[s], [zeros16, ulane])
        bi = plsc.load_gather(ibp[s], [zeros16, ilane])
        return part, bu, bi

    def finish(r, part, bu, bi):
        for k in (1, 2, 4, 8):
            perm = lax.gather(
                part, (iota16 ^ k)[:, None],
                dimension_numbers=lax.GatherDimensionNumbers(
                    offset_dims=(), collapsed_slice_dims=(0,),
                    start_index_map=(0,)),
                slice_sizes=(1,),
                mode=lax.GatherScatterMode.PROMISE_IN_BOUNDS)
            part = part + perm
        val = part + bu + bi + _MU2
        gbase = (r >> 4) << 4
        mask = iota16 == (r & 15)
        sl = pl.ds(gbase, _L)
        out_v[sl] = out_v[sl] + jnp.where(mask, val,
                                          jnp.zeros((_L,), jnp.float32))

    for s in range(_D):
        fetch(s, s)

    def step(g, c):
        for s in range(_D):
            r = _D * g + s
            wait(s)
            part, bu, bi = extract(r, s)

            @pl.when(r + _D < _BPW)
            def _():
                fetch(r + _D, s)

            finish(r, part, bu, bi)

        return c

    _MAIN = _BPW // _D
    lax.fori_loop(0, _MAIN, step, 0)
    for s in range(_BPW - _MAIN * _D):
        r = _MAIN * _D + s
        wait(s)
        part, bu, bi = extract(r, s)
        finish(r, part, bu, bi)
    pltpu.sync_copy(out_v, out_hbm.at[pl.ds(base, _BPW)])


def kernel(user_indices, item_indices, user_emb, item_emb, user_bias,
           item_bias):
    mesh = plsc.VectorSubcoreMesh(core_axis_name="c", subcore_axis_name="s")
    pan = pltpu.VMEM((_LATENT, 128), jnp.float32)
    bp = pltpu.VMEM((1, 128), jnp.float32)
    sem = pltpu.SemaphoreType.DMA
    f = pl.kernel(
        _body,
        out_type=jax.ShapeDtypeStruct((_B,), jnp.float32),
        mesh=mesh,
        compiler_params=pltpu.CompilerParams(
            needs_layout_passes=False, use_tc_tiling_on_sc=True),
        scratch_types=(
            [pltpu.VMEM((_IPAD,), jnp.int32)] * 2
            + [pan] * 8 + [bp] * 8
            + [pltpu.VMEM((_BPW,), jnp.float32)]
            + [sem] * 12
        ),
    )
    return f(user_indices.astype(jnp.int32), item_indices.astype(jnp.int32),
             user_emb.T, item_emb.T, user_bias.T, item_bias.T)


# speedup vs baseline: 1.0079x; 1.0079x over previous
import jax
import jax.numpy as jnp
from jax import lax
from jax.experimental import pallas as pl
from jax.experimental.pallas import tpu as pltpu
from jax.experimental.pallas import tpu_sc as plsc

_LATENT = 64
_MU2 = 7.0
_B = 16384

_info = plsc.get_sparse_core_info()
_NC = _info.num_cores
_NS = _info.num_subcores
_NW = _NC * _NS
_BPW = _B // _NW
_L = _info.num_lanes
_IPAD = _BPW + _L
_D = 4


def _body(uidx_hbm, iidx_hbm, uembt_hbm, iembt_hbm, ubt_hbm, ibt_hbm,
          out_hbm,
          uidx_v, iidx_v,
          up0, up1, up2, up3, ip0, ip1, ip2, ip3,
          ub0, ub1, ub2, ub3, ib0, ib1, ib2, ib3,
          out_v,
          su0, su1, su2, su3, si0, si1, si2, si3,
          sb0, sb1, sb2, sb3):
    wid = lax.axis_index("s") * _NC + lax.axis_index("c")
    base = wid * _BPW

    upan = (up0, up1, up2, up3)
    ipan = (ip0, ip1, ip2, ip3)
    ubp = (ub0, ub1, ub2, ub3)
    ibp = (ib0, ib1, ib2, ib3)
    su = (su0, su1, su2, su3)
    si = (si0, si1, si2, si3)
    sb = (sb0, sb1, sb2, sb3)

    pltpu.sync_copy(uidx_hbm.at[pl.ds(base, _BPW)],
                    uidx_v.at[pl.ds(0, _BPW)])
    pltpu.sync_copy(iidx_hbm.at[pl.ds(base, _BPW)],
                    iidx_v.at[pl.ds(0, _BPW)])

    iota16 = lax.iota(jnp.int32, _L)
    zeros16 = jnp.zeros((_L,), jnp.int32)

    def zinit(g, c):
        out_v[pl.ds(g * _L, _L)] = jnp.zeros((_L,), jnp.float32)
        return c

    lax.fori_loop(0, _BPW // _L, zinit, 0)

    def fetch(r, s):
        ui = uidx_v[pl.ds(r, _L)][0]
        ii = iidx_v[pl.ds(r, _L)][0]
        uoff = pl.multiple_of((ui >> 7) * 128, 128)
        ioff = pl.multiple_of((ii >> 7) * 128, 128)
        pltpu.async_copy(uembt_hbm.at[:, pl.ds(uoff, 128)], upan[s], su[s])
        pltpu.async_copy(iembt_hbm.at[:, pl.ds(ioff, 128)], ipan[s], si[s])
        pltpu.async_copy(ubt_hbm.at[:, pl.ds(uoff, 128)], ubp[s], sb[s])
        pltpu.async_copy(ibt_hbm.at[:, pl.ds(ioff, 128)], ibp[s], sb[s])

    def wait(s):
        pltpu.make_async_copy(uembt_hbm.at[:, pl.ds(0, 128)], upan[s],
                              su[s]).wait()
        pltpu.make_async_copy(iembt_hbm.at[:, pl.ds(0, 128)], ipan[s],
                              si[s]).wait()
        pltpu.make_async_copy(ubt_hbm.at[:, pl.ds(0, 128)], ubp[s],
                              sb[s]).wait()
        pltpu.make_async_copy(ibt_hbm.at[:, pl.ds(0, 128)], ibp[s],
                              sb[s]).wait()

    def compute(r, s):
        ui = uidx_v[pl.ds(r, _L)][0]
        ii = iidx_v[pl.ds(r, _L)][0]
        ulane = zeros16 + (ui & 127)
        ilane = zeros16 + (ii & 127)
        part = jnp.zeros((_L,), jnp.float32)
        for c in range(_LATENT // _L):
            dvec = c * _L + iota16
            du = plsc.load_gather(upan[s], [dvec, ulane])
            di = plsc.load_gather(ipan[s], [dvec, ilane])
            part = part + du * di
        for k in (1, 2, 4, 8):
            perm = lax.gather(
                part, (iota16 ^ k)[:, None],
                dimension_numbers=lax.GatherDimensionNumbers(
                    offset_dims=(), collapsed_slice_dims=(0,),
                    start_index_map=(0,)),
                slice_sizes=(1,),
                mode=lax.GatherScatterMode.PROMISE_IN_BOUNDS)
            part = part + perm
        bu = plsc.load_gather(ubp[s], [zeros16, ulane])
        bi = plsc.load_gather(ibp[s], [zeros16, ilane])
        val = part + bu + bi + _MU2
        gbase = (r >> 4) << 4
        mask = iota16 == (r & 15)
        sl = pl.ds(gbase, _L)
        out_v[sl] = out_v[sl] + jnp.where(mask, val,
                                          jnp.zeros((_L,), jnp.float32))

    for s in range(_D):
        fetch(s, s)

    def step(g, c):
        for s in range(_D):
            r = _D * g + s
            wait(s)
            compute(r, s)

            @pl.when(r + _D < _BPW)
            def _():
                fetch(r + _D, s)

        return c

    _MAIN = _BPW // _D
    lax.fori_loop(0, _MAIN, step, 0)
    for s in range(_BPW - _MAIN * _D):
        wait(s)
        compute(_MAIN * _D + s, s)
    pltpu.sync_copy(out_v, out_hbm.at[pl.ds(base, _BPW)])


def kernel(user_indices, item_indices, user_emb, item_emb, user_bias,
           item_bias):
    mesh = plsc.VectorSubcoreMesh(core_axis_name="c", subcore_axis_name="s")
    pan = pltpu.VMEM((_LATENT, 128), jnp.float32)
    bp = pltpu.VMEM((1, 128), jnp.float32)
    sem = pltpu.SemaphoreType.DMA
    f = pl.kernel(
        _body,
        out_type=jax.ShapeDtypeStruct((_B,), jnp.float32),
        mesh=mesh,
        compiler_params=pltpu.CompilerParams(
            needs_layout_passes=False, use_tc_tiling_on_sc=True),
        scratch_types=(
            [pltpu.VMEM((_IPAD,), jnp.int32)] * 2
            + [pan] * 8 + [bp] * 8
            + [pltpu.VMEM((_BPW,), jnp.float32)]
            + [sem] * 12
        ),
    )
    return f(user_indices.astype(jnp.int32), item_indices.astype(jnp.int32),
             user_emb.T, item_emb.T, user_bias.T, item_bias.T)
